# Initial kernel scaffold; baseline (speedup 1.0000x reference)
import jax
import jax.numpy as jnp
from jax.experimental import pallas as pl


def kernel(risk_scores_sequence, edge_to_complex, h_final, node_to_complex, num_complexes, hz_W1, hz_b1, hz_W2, hz_b2, ko_W1, ko_b1, ko_g1, ko_bn1, ko_W2, ko_b2, ko_g2, ko_bn2, ko_W3, ko_b3):
    raise NotImplementedError("write your pallas kernel here")


# speedup vs baseline: 10.6163x; 10.6163x over previous
import functools

import jax
import jax.numpy as jnp
from jax import lax
from jax.experimental import pallas as pl
from jax.experimental.pallas import tpu as pltpu
from jax.experimental.pallas import tpu_sc as plsc

T = 8
E = 800000
N = 50000
D = 256
C = 4096

NC = 2
NS = 16
NW = NC * NS

CP = 4104
ACC_W = CP * 16
CNT_W = CP * 8
NPAD = 50176
RPC = NPAD // NC
RCH = 512
NCH = RPC // RCH

EPW = E // NW
EBIN_W = T * C
EWINS = ((0, 12496), (12496, 12504))

_mesh = plsc.VectorSubcoreMesh(core_axis_name="c", subcore_axis_name="s",
                               num_cores=NC, num_subcores=NS)
_sc_params = pltpu.CompilerParams(needs_layout_passes=False)


def _zero_fill(ref, n):
    zeros16 = jnp.zeros((16,), jnp.float32)

    def body(i, _):
        ref[pl.ds(i * 16, 16)] = zeros16
        return ()
    lax.fori_loop(0, n // 16, body, ())


def _node_body(hflat_hbm, n2c_hbm, out_acc, out_cnt, acc_v, cnt_v, row_v, nidx_v):
    cid = lax.axis_index("c")
    sid = lax.axis_index("s")

    _zero_fill(acc_v, ACC_W)
    _zero_fill(cnt_v, CNT_W)

    iota16 = lax.iota(jnp.int32, 16)
    ones16 = jnp.ones((16,), jnp.float32)
    lane0 = iota16 == 0

    def chunk(ch, _):
        off = cid * RPC + ch * RCH
        pltpu.sync_copy(hflat_hbm.at[pl.ds(sid * (NPAD * 16) + off * 16,
                                           RCH * 16)], row_v)
        pltpu.sync_copy(n2c_hbm.at[pl.ds(off, RCH)], nidx_v)

        def row(r, _):
            rsplat = jnp.zeros((16,), jnp.int32) + r
            seg = plsc.load_gather(nidx_v, [rsplat])
            val = row_v[pl.ds(r * 16, 16)]
            plsc.addupdate_scatter(acc_v, [seg * 16 + iota16], val)
            caddr = seg * 8 + (r % 8)
            plsc.addupdate_scatter(cnt_v, [caddr], ones16, mask=lane0)
            return ()
        lax.fori_loop(0, RCH, row, ())
        return ()
    lax.fori_loop(0, NCH, chunk, ())

    w = cid * NS + sid
    pltpu.sync_copy(acc_v, out_acc.at[pl.ds(w * ACC_W, ACC_W)])
    pltpu.sync_copy(cnt_v, out_cnt.at[pl.ds(w * CNT_W, CNT_W)])


_node_call = pl.kernel(
    _node_body,
    out_type=[
        jax.ShapeDtypeStruct((NW * ACC_W,), jnp.float32),
        jax.ShapeDtypeStruct((NW * CNT_W,), jnp.float32),
    ],
    mesh=_mesh,
    compiler_params=_sc_params,
    scratch_types=[
        pltpu.VMEM((ACC_W,), jnp.float32),
        pltpu.VMEM((CNT_W,), jnp.float32),
        pltpu.VMEM((RCH * 16,), jnp.float32),
        pltpu.VMEM((RCH,), jnp.int32),
    ],
)


def _edge_body(risk_hbm, e2c_hbm, out_bins, sum_v, cnt_v, idx_v, val_v):
    cid = lax.axis_index("c")
    sid = lax.axis_index("s")
    w = cid * NS + sid

    _zero_fill(sum_v, EBIN_W)
    _zero_fill(cnt_v, EBIN_W)

    iota16 = lax.iota(jnp.int32, 16)
    ones16 = jnp.ones((16,), jnp.float32)

    ebase = w * EPW
    for t in range(T):
        for woff, wsz in EWINS:
            base = t * E + ebase + woff
            pltpu.sync_copy(e2c_hbm.at[pl.ds(base, wsz)],
                            idx_v.at[pl.ds(0, wsz)])
            pltpu.sync_copy(risk_hbm.at[pl.ds(base, wsz)],
                            val_v.at[pl.ds(0, wsz)])
            nv = (wsz + 15) // 16

            def vloop(i, _, t=t, wsz=wsz):
                idxv = idx_v[pl.ds(i * 16, 16)]
                valv = val_v[pl.ds(i * 16, 16)]
                mask = iota16 < (wsz - i * 16)
                addr = idxv + t * C
                plsc.addupdate_scatter(sum_v, [addr], valv, mask=mask)
                plsc.addupdate_scatter(cnt_v, [addr], ones16, mask=mask)
                return ()
            lax.fori_loop(0, nv, vloop, ())

    pltpu.sync_copy(sum_v, out_bins.at[pl.ds(w * 2 * EBIN_W, EBIN_W)])
    pltpu.sync_copy(cnt_v, out_bins.at[pl.ds(w * 2 * EBIN_W + EBIN_W, EBIN_W)])


_edge_call = pl.kernel(
    _edge_body,
    out_type=[jax.ShapeDtypeStruct((NW * 2 * EBIN_W,), jnp.float32)],
    mesh=_mesh,
    compiler_params=_sc_params,
    scratch_types=[
        pltpu.VMEM((EBIN_W,), jnp.float32),
        pltpu.VMEM((EBIN_W,), jnp.float32),
        pltpu.VMEM((12512,), jnp.int32),
        pltpu.VMEM((12512,), jnp.float32),
    ],
)


def _tc_body(acc, cnt, ebins,
             hzW1c_r, hzb1c_r, hzW2r_r, hzb2_r,
             koW1_r, kob1_r, kog1_r, kobn1_r, koW2_r, kob2_r, kog2_r,
             kobn2_r, koW3_r, kob3_r,
             lk_out, hz_out, sv_out):
    hzW1c, hzb1c, hzW2r, hzb2 = (hzW1c_r[...], hzb1c_r[...], hzW2r_r[...],
                                 hzb2_r[...])
    koW1, kob1, kog1, kobn1 = koW1_r[...], kob1_r[...], kog1_r[...], kobn1_r[...]
    koW2, kob2, kog2, kobn2 = koW2_r[...], kob2_r[...], kog2_r[...], kobn2_r[...]
    koW3, kob3 = koW3_r[...], kob3_r[...]

    ns = acc[0, :C, :] + acc[1, :C, :]
    denom = jnp.sum(cnt[:C, :], axis=1, keepdims=True) * (1.0 / NS)
    cf = ns / jnp.clip(denom, 1.0, None)

    def ln(x, g, b):
        mu = jnp.mean(x, axis=-1, keepdims=True)
        d = x - mu
        var = jnp.mean(d * d, axis=-1, keepdims=True)
        return d / jnp.sqrt(var + 1e-5) * g + b

    dot = functools.partial(jnp.dot, preferred_element_type=jnp.float32)
    h = jax.nn.relu(ln(dot(cf, koW1) + kob1, kog1, kobn1))
    h = jax.nn.relu(ln(dot(h, koW2) + kob2, kog2, kobn2))
    lk_out[...] = dot(h, koW3) + kob3

    eb = jnp.sum(ebins[...], axis=0)
    es = eb[0]
    ec = eb[1]
    m = es / jnp.clip(ec, 1.0, None)

    lam_rows = []
    sv_rows = []
    cum = None
    for t in range(T):
        mt = lax.slice_in_dim(m, t, t + 1, axis=0)
        hh = jax.nn.relu(hzW1c * mt + hzb1c)
        lam = jax.nn.sigmoid(dot(hzW2r, hh) + hzb2)
        lam_rows.append(lam)
        step = jnp.clip(1.0 - lam, 1e-7, 1.0)
        cum = step if cum is None else cum * step
        sv_rows.append(cum)
    hz_out[...] = jnp.concatenate(lam_rows, axis=0)
    sv_out[...] = jnp.concatenate(sv_rows, axis=0)


_tc_call = pl.pallas_call(
    _tc_body,
    out_shape=[
        jax.ShapeDtypeStruct((C, 1), jnp.float32),
        jax.ShapeDtypeStruct((T, C), jnp.float32),
        jax.ShapeDtypeStruct((T, C), jnp.float32),
    ],
)


def kernel(risk_scores_sequence, edge_to_complex, h_final, node_to_complex,
           num_complexes,
           hz_W1, hz_b1, hz_W2, hz_b2,
           ko_W1, ko_b1, ko_g1, ko_bn1, ko_W2, ko_b2, ko_g2, ko_bn2,
           ko_W3, ko_b3):
    risk_flat = risk_scores_sequence.reshape(T * E)
    e2c_flat = edge_to_complex.reshape(T * E)
    h_pad = jnp.pad(h_final, ((0, NPAD - N), (0, 0)))
    h_shuf = h_pad.reshape(NPAD, NS, 16).transpose(1, 0, 2).reshape(-1)
    n2c_pad = jnp.pad(node_to_complex, (0, NPAD - N), constant_values=C)

    acc_flat, cnt_flat = _node_call(h_shuf, n2c_pad)
    (ebins_flat,) = _edge_call(risk_flat, e2c_flat)

    acc_t = (acc_flat.reshape(NC, NS, CP, 16).transpose(0, 2, 1, 3)
             .reshape(NC, CP, D))
    cnt_t = (cnt_flat.reshape(NC * NS, CP, 8).transpose(1, 0, 2)
             .reshape(CP, NC * NS * 8))
    lk, hz, sv = _tc_call(
        acc_t,
        cnt_t,
        ebins_flat.reshape(NW, 2, T, C),
        hz_W1.reshape(D, 1), hz_b1.reshape(D, 1),
        hz_W2.reshape(1, D), hz_b2.reshape(1, 1),
        ko_W1, ko_b1.reshape(1, D), ko_g1.reshape(1, D), ko_bn1.reshape(1, D),
        ko_W2, ko_b2.reshape(1, D), ko_g2.reshape(1, D), ko_bn2.reshape(1, D),
        ko_W3, ko_b3.reshape(1, 1))

    return (lk[:, 0], hz, sv)


# speedup vs baseline: 15.1195x; 1.4242x over previous
import functools

import jax
import jax.numpy as jnp
from jax import lax
from jax.experimental import pallas as pl
from jax.experimental.pallas import tpu as pltpu
from jax.experimental.pallas import tpu_sc as plsc

T = 8
E = 800000
N = 50000
D = 256
C = 4096

NC = 2
NS = 16
NW = NC * NS

CP = 4104
ACC_W = CP * 16
CNT_W = CP * 8
RPC = N // NC
RCH = 512
NFULL = RPC // RCH
RTAIL = RPC - NFULL * RCH

EBIN_W = T * C
EW = 3200
NWIN = E // EW

_mesh = plsc.VectorSubcoreMesh(core_axis_name="c", subcore_axis_name="s",
                               num_cores=NC, num_subcores=NS)
_sc_params = pltpu.CompilerParams(needs_layout_passes=False,
                                  use_tc_tiling_on_sc=False)


def _zero_fill(ref, n):
    zeros16 = jnp.zeros((16,), jnp.float32)

    def body(i, _):
        ref[pl.ds(i * 16, 16)] = zeros16
        return ()
    lax.fori_loop(0, n // 16, body, ())


def _node_body(h_hbm, n2c_hbm, out_acc, out_cnt, acc_v, cnt_v, row_v, nidx_v):
    cid = lax.axis_index("c")
    sid = lax.axis_index("s")

    _zero_fill(acc_v, ACC_W)
    _zero_fill(cnt_v, CNT_W)

    iota16 = lax.iota(jnp.int32, 16)
    ones16 = jnp.ones((16,), jnp.float32)
    lane0 = iota16 == 0
    zeros16i = jnp.zeros((16,), jnp.int32)

    def do_chunk(off, ch_rows):
        pltpu.sync_copy(h_hbm.at[pl.ds(off, ch_rows), pl.ds(sid * 16, 16)],
                        row_v.at[pl.ds(0, ch_rows), :])
        pltpu.sync_copy(n2c_hbm.at[pl.ds(off, ch_rows)],
                        nidx_v.at[pl.ds(0, ch_rows)])

        def rloop(r4, _):
            for u in range(4):
                r = r4 * 4 + u
                seg = plsc.load_gather(nidx_v, [zeros16i + r])
                val = row_v[r, :]
                plsc.addupdate_scatter(acc_v, [seg * 16 + iota16], val)
                plsc.addupdate_scatter(cnt_v, [seg * 8 + (r % 8)], ones16,
                                       mask=lane0)
            return ()
        lax.fori_loop(0, ch_rows // 4, rloop, ())

    def chunk(ch, _):
        do_chunk(cid * RPC + ch * RCH, RCH)
        return ()
    lax.fori_loop(0, NFULL, chunk, ())
    do_chunk(cid * RPC + NFULL * RCH, RTAIL)

    w = cid * NS + sid
    pltpu.sync_copy(acc_v, out_acc.at[pl.ds(w * ACC_W, ACC_W)])
    pltpu.sync_copy(cnt_v, out_cnt.at[pl.ds(w * CNT_W, CNT_W)])


_node_call = pl.kernel(
    _node_body,
    out_type=[
        jax.ShapeDtypeStruct((NW * ACC_W,), jnp.float32),
        jax.ShapeDtypeStruct((NW * CNT_W,), jnp.float32),
    ],
    mesh=_mesh,
    compiler_params=_sc_params,
    scratch_types=[
        pltpu.VMEM((ACC_W,), jnp.float32),
        pltpu.VMEM((CNT_W,), jnp.float32),
        pltpu.VMEM((RCH, 16), jnp.float32),
        pltpu.VMEM((RCH,), jnp.int32),
    ],
)


def _edge_body(risk_hbm, e2c_hbm, out_bins, sum_v, cnt_v, idx_v, val_v):
    cid = lax.axis_index("c")
    sid = lax.axis_index("s")
    w = cid * NS + sid

    _zero_fill(sum_v, EBIN_W)
    _zero_fill(cnt_v, EBIN_W)

    ones16 = jnp.ones((16,), jnp.float32)

    nwin = jnp.where(w < 26, 8, 7)
    lo = w * 7 + jnp.minimum(w, 26)

    def win(k, _):
        col0 = (lo + k) * EW
        pltpu.sync_copy(e2c_hbm.at[:, pl.ds(col0, EW)], idx_v)
        pltpu.sync_copy(risk_hbm.at[:, pl.ds(col0, EW)], val_v)
        for t in range(T):
            def vloop(i, _, t=t):
                for u in range(8):
                    o = (i * 8 + u) * 16
                    idxv = idx_v[t, pl.ds(o, 16)]
                    valv = val_v[t, pl.ds(o, 16)]
                    addr = idxv + t * C
                    plsc.addupdate_scatter(sum_v, [addr], valv)
                    plsc.addupdate_scatter(cnt_v, [addr], ones16)
                return ()
            lax.fori_loop(0, EW // 128, vloop, ())
        return ()
    lax.fori_loop(0, nwin, win, ())

    pltpu.sync_copy(sum_v, out_bins.at[pl.ds(w * 2 * EBIN_W, EBIN_W)])
    pltpu.sync_copy(cnt_v, out_bins.at[pl.ds(w * 2 * EBIN_W + EBIN_W, EBIN_W)])


_edge_call = pl.kernel(
    _edge_body,
    out_type=[jax.ShapeDtypeStruct((NW * 2 * EBIN_W,), jnp.float32)],
    mesh=_mesh,
    compiler_params=_sc_params,
    scratch_types=[
        pltpu.VMEM((EBIN_W,), jnp.float32),
        pltpu.VMEM((EBIN_W,), jnp.float32),
        pltpu.VMEM((T, EW), jnp.int32),
        pltpu.VMEM((T, EW), jnp.float32),
    ],
)


def _tc_body(acc, cnt, ebins,
             hzW1c_r, hzb1c_r, hzW2r_r, hzb2_r,
             koW1_r, kob1_r, kog1_r, kobn1_r, koW2_r, kob2_r, kog2_r,
             kobn2_r, koW3_r, kob3_r,
             lk_out, hz_out, sv_out):
    hzW1c, hzb1c, hzW2r, hzb2 = (hzW1c_r[...], hzb1c_r[...], hzW2r_r[...],
                                 hzb2_r[...])
    koW1, kob1, kog1, kobn1 = koW1_r[...], kob1_r[...], kog1_r[...], kobn1_r[...]
    koW2, kob2, kog2, kobn2 = koW2_r[...], kob2_r[...], kog2_r[...], kobn2_r[...]
    koW3, kob3 = koW3_r[...], kob3_r[...]

    ns = acc[0, :C, :] + acc[1, :C, :]
    denom = jnp.sum(cnt[:C, :], axis=1, keepdims=True) * (1.0 / NS)
    cf = ns / jnp.clip(denom, 1.0, None)

    def ln(x, g, b):
        mu = jnp.mean(x, axis=-1, keepdims=True)
        d = x - mu
        var = jnp.mean(d * d, axis=-1, keepdims=True)
        return d / jnp.sqrt(var + 1e-5) * g + b

    dot = functools.partial(jnp.dot, preferred_element_type=jnp.float32)
    h = jax.nn.relu(ln(dot(cf, koW1) + kob1, kog1, kobn1))
    h = jax.nn.relu(ln(dot(h, koW2) + kob2, kog2, kobn2))
    lk_out[...] = dot(h, koW3) + kob3

    eb = jnp.sum(ebins[...], axis=0)
    es = eb[0]
    ec = eb[1]
    m = es / jnp.clip(ec, 1.0, None)

    lam_rows = []
    sv_rows = []
    cum = None
    for t in range(T):
        mt = lax.slice_in_dim(m, t, t + 1, axis=0)
        hh = jax.nn.relu(hzW1c * mt + hzb1c)
        lam = jax.nn.sigmoid(dot(hzW2r, hh) + hzb2)
        lam_rows.append(lam)
        step = jnp.clip(1.0 - lam, 1e-7, 1.0)
        cum = step if cum is None else cum * step
        sv_rows.append(cum)
    hz_out[...] = jnp.concatenate(lam_rows, axis=0)
    sv_out[...] = jnp.concatenate(sv_rows, axis=0)


_tc_call = pl.pallas_call(
    _tc_body,
    out_shape=[
        jax.ShapeDtypeStruct((C, 1), jnp.float32),
        jax.ShapeDtypeStruct((T, C), jnp.float32),
        jax.ShapeDtypeStruct((T, C), jnp.float32),
    ],
)


def kernel(risk_scores_sequence, edge_to_complex, h_final, node_to_complex,
           num_complexes,
           hz_W1, hz_b1, hz_W2, hz_b2,
           ko_W1, ko_b1, ko_g1, ko_bn1, ko_W2, ko_b2, ko_g2, ko_bn2,
           ko_W3, ko_b3):
    risk2 = risk_scores_sequence[:, :, 0]

    acc_flat, cnt_flat = _node_call(h_final, node_to_complex)
    (ebins_flat,) = _edge_call(risk2, edge_to_complex)

    acc_t = (acc_flat.reshape(NC, NS, CP, 16).transpose(0, 2, 1, 3)
             .reshape(NC, CP, D))
    cnt_t = (cnt_flat.reshape(NC * NS, CP, 8).transpose(1, 0, 2)
             .reshape(CP, NC * NS * 8))
    lk, hz, sv = _tc_call(
        acc_t,
        cnt_t,
        ebins_flat.reshape(NW, 2, T, C),
        hz_W1.reshape(D, 1), hz_b1.reshape(D, 1),
        hz_W2.reshape(1, D), hz_b2.reshape(1, 1),
        ko_W1, ko_b1.reshape(1, D), ko_g1.reshape(1, D), ko_bn1.reshape(1, D),
        ko_W2, ko_b2.reshape(1, D), ko_g2.reshape(1, D), ko_bn2.reshape(1, D),
        ko_W3, ko_b3.reshape(1, 1))

    return (lk[:, 0], hz, sv)
